# Initial kernel scaffold; baseline (speedup 1.0000x reference)
import jax
import jax.numpy as jnp
from jax.experimental import pallas as pl


def kernel(x, fc_weight, bias):
    raise NotImplementedError("write your pallas kernel here")


# speedup vs baseline: 1.4458x; 1.4458x over previous
import functools

import jax
import jax.numpy as jnp
from jax import lax
from jax.experimental import pallas as pl
from jax.experimental.pallas import tpu as pltpu
from jax.experimental.pallas import tpu_sc as plsc

_NC = 2
_NS = 16
_NW = _NC * _NS
_LANES = 16


def _sc_embed_sum(idx_t, table_flat, bias16, batch, num_fields):
    b_per_w = batch // _NW
    n_idx = num_fields * b_per_w
    n_vec = b_per_w // _LANES
    mesh = plsc.VectorSubcoreMesh(core_axis_name="c", subcore_axis_name="s")

    @functools.partial(
        pl.kernel,
        out_type=jax.ShapeDtypeStruct((batch,), jnp.float32),
        mesh=mesh,
        scratch_types=[
            pltpu.VMEM((n_idx,), jnp.int32),
            pltpu.VMEM((n_idx,), jnp.float32),
            pltpu.VMEM((b_per_w,), jnp.float32),
            pltpu.VMEM((_LANES,), jnp.float32),
            pltpu.SemaphoreType.DMA,
        ],
    )
    def k(idx_hbm, table_hbm, bias_hbm, out_hbm, idx_v, vals_v, out_v, bias_v, sem):
        wid = lax.axis_index("s") * _NC + lax.axis_index("c")
        base = wid * b_per_w
        pltpu.sync_copy(bias_hbm, bias_v)
        pltpu.sync_copy(idx_hbm.at[wid], idx_v)
        pltpu.async_copy(table_hbm.at[idx_v], vals_v, sem).wait()

        def body(v, carry):
            acc = bias_v[...]
            for f in range(num_fields):
                acc = acc + vals_v[pl.ds(f * b_per_w + v * _LANES, _LANES)]
            out_v[pl.ds(v * _LANES, _LANES)] = acc
            return carry

        lax.fori_loop(0, n_vec, body, 0)
        pltpu.sync_copy(out_v, out_hbm.at[pl.ds(base, b_per_w)])

    return k(idx_t, table_flat, bias16)


def kernel(x, fc_weight, bias):
    batch, num_fields = x.shape
    b_per_w = batch // _NW
    idx_t = (
        x.reshape(_NW, b_per_w, num_fields)
        .transpose(0, 2, 1)
        .reshape(_NW, num_fields * b_per_w)
    )
    table_flat = fc_weight.reshape(-1)
    bias16 = jnp.broadcast_to(bias, (_LANES,))
    out = _sc_embed_sum(idx_t, table_flat, bias16, batch, num_fields)
    return out.reshape(batch, fc_weight.shape[1])
